# baseline (device time: 13356 ns/iter reference)
import os

import jax
import jax.numpy as jnp
from jax import lax
from jax.experimental import pallas as pl
from jax.experimental.pallas import tpu as pltpu

N_DEV = 4
N_CHUNKS = 8

_VARIANT = os.environ.get("KERNEL_VARIANT", "full")


def kernel(x):
    m_per, n = x.shape
    rows = m_per // N_CHUNKS

    def body(x_ref, out_ref, chunk_ref, gather_ref, copy_sems,
             send_sems, recv_sems):
        me = lax.axis_index("i")

        copies = []
        for c in range(N_CHUNKS):
            copies.append(pltpu.make_async_copy(
                x_ref.at[pl.ds(c * rows, rows), :],
                chunk_ref.at[c],
                copy_sems.at[c],
            ))
        for cp in copies:
            cp.start()

        barrier_sem = pltpu.get_barrier_semaphore()
        for d in range(1, N_DEV):
            peer = lax.rem(me + d, N_DEV)
            pl.semaphore_signal(
                barrier_sem, inc=1,
                device_id=(peer,), device_id_type=pl.DeviceIdType.MESH,
            )
        pl.semaphore_wait(barrier_sem, N_DEV - 1)

        acc = None
        for c in range(N_CHUNKS):
            copies[c].wait()
            if _VARIANT == "noreduce" and c > 0:
                continue
            part = jnp.max(chunk_ref[c], axis=0, keepdims=True)
            acc = part if acc is None else jnp.maximum(acc, part)
        gather_ref[0, :, :] = acc

        if _VARIANT != "full":
            out_ref[:, :] = gather_ref[0, :, :]
            return

        rdmas = []
        for d in range(1, N_DEV):
            peer = lax.rem(me + d, N_DEV)
            rdma = pltpu.make_async_remote_copy(
                src_ref=gather_ref.at[0],
                dst_ref=gather_ref.at[d],
                send_sem=send_sems.at[d - 1],
                recv_sem=recv_sems.at[d - 1],
                device_id=(peer,),
                device_id_type=pl.DeviceIdType.MESH,
            )
            rdma.start()
            rdmas.append(rdma)
        for rdma in rdmas:
            rdma.wait_recv()

        out_ref[:, :] = jnp.max(gather_ref[:, 0, :], axis=0, keepdims=True)

        for rdma in rdmas:
            rdma.wait_send()

    return pl.pallas_call(
        body,
        out_shape=jax.ShapeDtypeStruct((1, n), x.dtype),
        in_specs=[pl.BlockSpec(memory_space=pl.ANY)],
        out_specs=pl.BlockSpec(memory_space=pltpu.VMEM),
        scratch_shapes=[
            pltpu.VMEM((N_CHUNKS, rows, n), x.dtype),
            pltpu.VMEM((N_DEV, 1, n), x.dtype),
            pltpu.SemaphoreType.DMA((N_CHUNKS,)),
            pltpu.SemaphoreType.DMA((N_DEV - 1,)),
            pltpu.SemaphoreType.DMA((N_DEV - 1,)),
        ],
        compiler_params=pltpu.CompilerParams(collective_id=0),
    )(x)


# device time: 8430 ns/iter; 1.5843x vs baseline; 1.5843x over previous
import os

import jax
import jax.numpy as jnp
from jax import lax
from jax.experimental import pallas as pl
from jax.experimental.pallas import tpu as pltpu

N_DEV = 4
N_CHUNKS = 8

_VARIANT = os.environ.get("KERNEL_VARIANT", "full")


def kernel(x):
    m_per, n = x.shape
    rows = m_per // N_CHUNKS

    def body(x_ref, out_ref, chunk_ref, gather_ref, copy_sems,
             send_sems, recv_sems):
        me = lax.axis_index("i")

        copies = []
        for c in range(N_CHUNKS):
            copies.append(pltpu.make_async_copy(
                x_ref.at[pl.ds(c * rows, rows), :],
                chunk_ref.at[c],
                copy_sems.at[c],
            ))
        for cp in copies:
            cp.start()

        if _VARIANT != "local":
            barrier_sem = pltpu.get_barrier_semaphore()
            for d in range(1, N_DEV):
                peer = lax.rem(me + d, N_DEV)
                pl.semaphore_signal(
                    barrier_sem, inc=1,
                    device_id=(peer,), device_id_type=pl.DeviceIdType.MESH,
                )
            pl.semaphore_wait(barrier_sem, N_DEV - 1)

        acc = None
        for c in range(N_CHUNKS):
            copies[c].wait()
            if _VARIANT == "noreduce" and c > 0:
                continue
            part = jnp.max(chunk_ref[c], axis=0, keepdims=True)
            acc = part if acc is None else jnp.maximum(acc, part)
        gather_ref[0, :, :] = acc

        if _VARIANT != "full":
            out_ref[:, :] = gather_ref[0, :, :]
            return

        rdmas = []
        for d in range(1, N_DEV):
            peer = lax.rem(me + d, N_DEV)
            rdma = pltpu.make_async_remote_copy(
                src_ref=gather_ref.at[0],
                dst_ref=gather_ref.at[d],
                send_sem=send_sems.at[d - 1],
                recv_sem=recv_sems.at[d - 1],
                device_id=(peer,),
                device_id_type=pl.DeviceIdType.MESH,
            )
            rdma.start()
            rdmas.append(rdma)
        for rdma in rdmas:
            rdma.wait_recv()

        out_ref[:, :] = jnp.max(gather_ref[:, 0, :], axis=0, keepdims=True)

        for rdma in rdmas:
            rdma.wait_send()

    return pl.pallas_call(
        body,
        out_shape=jax.ShapeDtypeStruct((1, n), x.dtype),
        in_specs=[pl.BlockSpec(memory_space=pl.ANY)],
        out_specs=pl.BlockSpec(memory_space=pltpu.VMEM),
        scratch_shapes=[
            pltpu.VMEM((N_CHUNKS, rows, n), x.dtype),
            pltpu.VMEM((N_DEV, 1, n), x.dtype),
            pltpu.SemaphoreType.DMA((N_CHUNKS,)),
            pltpu.SemaphoreType.DMA((N_DEV - 1,)),
            pltpu.SemaphoreType.DMA((N_DEV - 1,)),
        ],
        compiler_params=(
            pltpu.CompilerParams()
            if _VARIANT == "local"
            else pltpu.CompilerParams(collective_id=0)
        ),
    )(x)
